# Initial kernel scaffold; baseline (speedup 1.0000x reference)
import jax
import jax.numpy as jnp
from jax.experimental import pallas as pl


def kernel(h, m, rbf, id_j, W_rbf, W1, W_res_0_0, W_res_0_1, W_res_1_0, W_res_1_1, scale_sum):
    raise NotImplementedError("write your pallas kernel here")


# speedup vs baseline: 1.5140x; 1.5140x over previous
import jax
import jax.numpy as jnp
import numpy as np
from jax.experimental import pallas as pl

SCALE_SILU = 1.0 / 0.6
INV_SQRT_2 = 1.0 / np.sqrt(2.0)

E_BLOCK = 2560
Q_TILE = 128
D = 128
R_BLOCK = 1264


def _ssilu(x):
    return jax.nn.sigmoid(x) * x * SCALE_SILU


def _dot(a, b):
    return jax.lax.dot_general(
        a, b, dimension_numbers=(((1,), (0,)), ((), ())),
        preferred_element_type=jnp.float32)


def _scatter_kernel(ids_ref, rbf_ref, m_ref, wrbf_ref, out_ref):
    i = pl.program_id(0)

    @pl.when(i == 0)
    def _zero():
        out_ref[...] = jnp.zeros_like(out_ref)

    mlp_rbf = _dot(rbf_ref[...], wrbf_ref[...])
    xm = m_ref[...] * mlp_rbf
    ids = ids_ref[0]
    q = ids // Q_TILE
    r = ids - q * Q_TILE
    col = jax.lax.broadcasted_iota(jnp.int32, (E_BLOCK, Q_TILE), 1)
    q_lo = q[0, 0]
    q_hi = q[E_BLOCK - 1, 0]

    def body(qq, carry):
        onehot = jnp.where((r == col) & (q == qq), 1.0, 0.0)
        part = jax.lax.dot_general(
            onehot, xm, dimension_numbers=(((0,), (0,)), ((), ())),
            preferred_element_type=jnp.float32)
        out_ref[pl.ds(qq * Q_TILE, Q_TILE), :] += part
        return carry

    jax.lax.fori_loop(q_lo, q_hi + 1, body, 0)


def _mlp_kernel(x2_ref, w1_ref, wa0_ref, wb0_ref, wa1_ref, wb1_ref, out_ref):
    x = _ssilu(_dot(x2_ref[...], w1_ref[...]))
    for wa, wb in ((wa0_ref, wb0_ref), (wa1_ref, wb1_ref)):
        y = _ssilu(_dot(x, wa[...]))
        y = _ssilu(_dot(y, wb[...]))
        x = (x + y) * INV_SQRT_2
    out_ref[...] = x


def kernel(h, m, rbf, id_j, W_rbf, W1, W_res_0_0, W_res_0_1, W_res_1_0,
           W_res_1_1, scale_sum):
    nAtoms = h.shape[0]
    E, d_edge = m.shape
    d_rbf = rbf.shape[1]
    nb = E // E_BLOCK
    n_pad = ((nAtoms + Q_TILE - 1) // Q_TILE) * Q_TILE

    ids3 = id_j.astype(jnp.int32).reshape(nb, E_BLOCK, 1)

    x2 = pl.pallas_call(
        _scatter_kernel,
        grid=(nb,),
        in_specs=[
            pl.BlockSpec((1, E_BLOCK, 1), lambda i: (i, 0, 0)),
            pl.BlockSpec((E_BLOCK, d_rbf), lambda i: (i, 0)),
            pl.BlockSpec((E_BLOCK, d_edge), lambda i: (i, 0)),
            pl.BlockSpec((d_rbf, d_edge), lambda i: (0, 0)),
        ],
        out_specs=pl.BlockSpec((n_pad, d_edge), lambda i: (0, 0)),
        out_shape=jax.ShapeDtypeStruct((n_pad, d_edge), jnp.float32),
    )(ids3, rbf, m, W_rbf)

    w1s = W1 * scale_sum

    out = pl.pallas_call(
        _mlp_kernel,
        grid=(n_pad // R_BLOCK,),
        in_specs=[pl.BlockSpec((R_BLOCK, d_edge), lambda i: (i, 0))]
        + [pl.BlockSpec((D, D), lambda i: (0, 0))] * 5,
        out_specs=pl.BlockSpec((R_BLOCK, D), lambda i: (i, 0)),
        out_shape=jax.ShapeDtypeStruct((n_pad, D), jnp.float32),
    )(x2, w1s, W_res_0_0, W_res_0_1, W_res_1_0, W_res_1_1)

    return out[:nAtoms]


# speedup vs baseline: 1.5186x; 1.0030x over previous
import jax
import jax.numpy as jnp
import numpy as np
from jax.experimental import pallas as pl

SCALE_SILU = 1.0 / 0.6
INV_SQRT_2 = 1.0 / np.sqrt(2.0)

E_BLOCK = 2560
Q_TILE = 128
D = 128
R_BLOCK = 1264


def _ssilu(x):
    return jax.nn.sigmoid(x) * x * SCALE_SILU


def _dot(a, b):
    return jax.lax.dot_general(
        a, b, dimension_numbers=(((1,), (0,)), ((), ())),
        preferred_element_type=jnp.float32)


def _scatter_kernel(ids_ref, rbf_ref, m_ref, wrbf_ref, out_ref):
    i = pl.program_id(0)

    @pl.when(i == 0)
    def _zero():
        out_ref[...] = jnp.zeros_like(out_ref)

    mlp_rbf = _dot(rbf_ref[...], wrbf_ref[...])
    xm = m_ref[...] * mlp_rbf
    ids = ids_ref[0]
    q = ids // Q_TILE
    r = ids - q * Q_TILE
    col = jax.lax.broadcasted_iota(jnp.int32, (E_BLOCK, Q_TILE), 1)
    q_lo = q[0, 0]
    q_hi = q[E_BLOCK - 1, 0]

    xm16 = xm.astype(jnp.bfloat16)

    def body(qq, carry):
        onehot = jnp.where((r == col) & (q == qq),
                           1.0, 0.0).astype(jnp.bfloat16)
        part = jax.lax.dot_general(
            onehot, xm16, dimension_numbers=(((0,), (0,)), ((), ())),
            preferred_element_type=jnp.float32)
        out_ref[pl.ds(qq * Q_TILE, Q_TILE), :] += part
        return carry

    jax.lax.fori_loop(q_lo, q_hi + 1, body, 0)


def _mlp_kernel(x2_ref, w1_ref, wa0_ref, wb0_ref, wa1_ref, wb1_ref, out_ref):
    x = _ssilu(_dot(x2_ref[...], w1_ref[...]))
    for wa, wb in ((wa0_ref, wb0_ref), (wa1_ref, wb1_ref)):
        y = _ssilu(_dot(x, wa[...]))
        y = _ssilu(_dot(y, wb[...]))
        x = (x + y) * INV_SQRT_2
    out_ref[...] = x


def kernel(h, m, rbf, id_j, W_rbf, W1, W_res_0_0, W_res_0_1, W_res_1_0,
           W_res_1_1, scale_sum):
    nAtoms = h.shape[0]
    E, d_edge = m.shape
    d_rbf = rbf.shape[1]
    nb = E // E_BLOCK
    n_pad = ((nAtoms + Q_TILE - 1) // Q_TILE) * Q_TILE

    ids3 = id_j.astype(jnp.int32).reshape(nb, E_BLOCK, 1)

    x2 = pl.pallas_call(
        _scatter_kernel,
        grid=(nb,),
        in_specs=[
            pl.BlockSpec((1, E_BLOCK, 1), lambda i: (i, 0, 0)),
            pl.BlockSpec((E_BLOCK, d_rbf), lambda i: (i, 0)),
            pl.BlockSpec((E_BLOCK, d_edge), lambda i: (i, 0)),
            pl.BlockSpec((d_rbf, d_edge), lambda i: (0, 0)),
        ],
        out_specs=pl.BlockSpec((n_pad, d_edge), lambda i: (0, 0)),
        out_shape=jax.ShapeDtypeStruct((n_pad, d_edge), jnp.float32),
    )(ids3, rbf, m, W_rbf)

    w1s = W1 * scale_sum

    out = pl.pallas_call(
        _mlp_kernel,
        grid=(n_pad // R_BLOCK,),
        in_specs=[pl.BlockSpec((R_BLOCK, d_edge), lambda i: (i, 0))]
        + [pl.BlockSpec((D, D), lambda i: (0, 0))] * 5,
        out_specs=pl.BlockSpec((R_BLOCK, D), lambda i: (i, 0)),
        out_shape=jax.ShapeDtypeStruct((n_pad, D), jnp.float32),
    )(x2, w1s, W_res_0_0, W_res_0_1, W_res_1_0, W_res_1_1)

    return out[:nAtoms]


# speedup vs baseline: 1.6491x; 1.0859x over previous
import jax
import jax.numpy as jnp
import numpy as np
from jax.experimental import pallas as pl
from jax.experimental.pallas import tpu as pltpu

SCALE_SILU = 1.0 / 0.6
INV_SQRT_2 = 1.0 / np.sqrt(2.0)

E_BLOCK = 2560
Q_TILE = 128
D = 128
R_BLOCK = 1264


def _ssilu(x):
    return jax.nn.sigmoid(x) * x * SCALE_SILU


def _dot(a, b):
    return jax.lax.dot_general(
        a, b, dimension_numbers=(((1,), (0,)), ((), ())),
        preferred_element_type=jnp.float32)


def _scatter_kernel(eb_ref, qt_ref, valid_ref, ids_ref, rbf_ref, m_ref,
                    wrbf_ref, out_ref):
    c = pl.program_id(0)

    @pl.when(c == 0)
    def _zero():
        out_ref[...] = jnp.zeros_like(out_ref)

    @pl.when(valid_ref[c] == 1)
    def _go():
        mlp_rbf = _dot(rbf_ref[...], wrbf_ref[...])
        xm16 = (m_ref[...] * mlp_rbf).astype(jnp.bfloat16)
        qq = qt_ref[c]
        rel = ids_ref[0] - qq * Q_TILE
        col = jax.lax.broadcasted_iota(jnp.int32, (E_BLOCK, Q_TILE), 1)
        onehot = jnp.where(rel == col, 1.0, 0.0).astype(jnp.bfloat16)
        part = jax.lax.dot_general(
            onehot, xm16, dimension_numbers=(((0,), (0,)), ((), ())),
            preferred_element_type=jnp.float32)
        out_ref[pl.ds(qq * Q_TILE, Q_TILE), :] += part


def _mlp_kernel(x2_ref, w1_ref, wa0_ref, wb0_ref, wa1_ref, wb1_ref, out_ref):
    x = _ssilu(_dot(x2_ref[...], w1_ref[...]))
    for wa, wb in ((wa0_ref, wb0_ref), (wa1_ref, wb1_ref)):
        y = _ssilu(_dot(x, wa[...]))
        y = _ssilu(_dot(y, wb[...]))
        x = (x + y) * INV_SQRT_2
    out_ref[...] = x


def kernel(h, m, rbf, id_j, W_rbf, W1, W_res_0_0, W_res_0_1, W_res_1_0,
           W_res_1_1, scale_sum):
    nAtoms = h.shape[0]
    E, d_edge = m.shape
    d_rbf = rbf.shape[1]
    nb = E // E_BLOCK
    n_pad = ((nAtoms + Q_TILE - 1) // Q_TILE) * Q_TILE
    nt = n_pad // Q_TILE

    ids32 = id_j.astype(jnp.int32)
    ids3 = ids32.reshape(nb, E_BLOCK, 1)

    tile_starts = jnp.searchsorted(
        ids32, jnp.arange(nt, dtype=jnp.int32) * Q_TILE).astype(jnp.int32)
    block_starts = jnp.arange(nb, dtype=jnp.int32) * E_BLOCK
    p = jnp.sort(jnp.concatenate([block_starts, tile_starts]))
    valid = (jnp.concatenate([jnp.ones((1,), jnp.bool_), p[1:] != p[:-1]])
             & (p < E)).astype(jnp.int32)
    pc = jnp.minimum(p, E - 1)
    eb = pc // E_BLOCK
    qt = ids32[pc] // Q_TILE

    grid_spec = pltpu.PrefetchScalarGridSpec(
        num_scalar_prefetch=3,
        grid=(nb + nt,),
        in_specs=[
            pl.BlockSpec((1, E_BLOCK, 1), lambda c, e, q, v: (e[c], 0, 0)),
            pl.BlockSpec((E_BLOCK, d_rbf), lambda c, e, q, v: (e[c], 0)),
            pl.BlockSpec((E_BLOCK, d_edge), lambda c, e, q, v: (e[c], 0)),
            pl.BlockSpec((d_rbf, d_edge), lambda c, e, q, v: (0, 0)),
        ],
        out_specs=pl.BlockSpec((n_pad, d_edge), lambda c, e, q, v: (0, 0)),
    )

    x2 = pl.pallas_call(
        _scatter_kernel,
        grid_spec=grid_spec,
        out_shape=jax.ShapeDtypeStruct((n_pad, d_edge), jnp.float32),
    )(eb, qt, valid, ids3, rbf, m, W_rbf)

    w1s = W1 * scale_sum

    out = pl.pallas_call(
        _mlp_kernel,
        grid=(n_pad // R_BLOCK,),
        in_specs=[pl.BlockSpec((R_BLOCK, d_edge), lambda i: (i, 0))]
        + [pl.BlockSpec((D, D), lambda i: (0, 0))] * 5,
        out_specs=pl.BlockSpec((R_BLOCK, D), lambda i: (i, 0)),
        out_shape=jax.ShapeDtypeStruct((n_pad, D), jnp.float32),
    )(x2, w1s, W_res_0_0, W_res_0_1, W_res_1_0, W_res_1_1)

    return out[:nAtoms]


# speedup vs baseline: 1.8806x; 1.1404x over previous
import jax
import jax.numpy as jnp
import numpy as np
from jax.experimental import pallas as pl
from jax.experimental.pallas import tpu as pltpu

SCALE_SILU = 1.0 / 0.6
INV_SQRT_2 = 1.0 / np.sqrt(2.0)

E_BLOCK = 2560
Q_TILE = 128
D = 128
R_BLOCK = 1264
LANES = 4


def _ssilu(x):
    return jax.nn.sigmoid(x) * x * SCALE_SILU


def _dot(a, b):
    return jax.lax.dot_general(
        a, b, dimension_numbers=(((1,), (0,)), ((), ())),
        preferred_element_type=jnp.float32)


def _scatter_kernel(qt_ref, v_ref, *refs):
    ids_refs = refs[0:LANES]
    rbf_refs = refs[LANES:2 * LANES]
    m_refs = refs[2 * LANES:3 * LANES]
    wrbf_ref = refs[3 * LANES]
    out_ref = refs[3 * LANES + 1]
    c = pl.program_id(0)

    @pl.when(c == 0)
    def _zero():
        out_ref[...] = jnp.zeros_like(out_ref)

    col = jax.lax.broadcasted_iota(jnp.int32, (E_BLOCK, Q_TILE), 1)
    for s in range(LANES):
        @pl.when(v_ref[s, c] == 1)
        def _go(s=s):
            mlp_rbf = _dot(rbf_refs[s][...], wrbf_ref[...])
            xm16 = (m_refs[s][...] * mlp_rbf).astype(jnp.bfloat16)
            qq = qt_ref[s, c]
            rel = ids_refs[s][0] - qq * Q_TILE
            onehot = jnp.where(rel == col, 1.0, 0.0).astype(jnp.bfloat16)
            part = jax.lax.dot_general(
                onehot, xm16, dimension_numbers=(((0,), (0,)), ((), ())),
                preferred_element_type=jnp.float32)
            out_ref[pl.ds(qq * Q_TILE, Q_TILE), :] += part


def _mlp_kernel(x2_ref, w1_ref, wa0_ref, wb0_ref, wa1_ref, wb1_ref, out_ref):
    x = _ssilu(_dot(x2_ref[...], w1_ref[...]))
    for wa, wb in ((wa0_ref, wb0_ref), (wa1_ref, wb1_ref)):
        y = _ssilu(_dot(x, wa[...]))
        y = _ssilu(_dot(y, wb[...]))
        x = (x + y) * INV_SQRT_2
    out_ref[...] = x


def kernel(h, m, rbf, id_j, W_rbf, W1, W_res_0_0, W_res_0_1, W_res_1_0,
           W_res_1_1, scale_sum):
    nAtoms = h.shape[0]
    E, d_edge = m.shape
    d_rbf = rbf.shape[1]
    nb = E // E_BLOCK
    n_pad = ((nAtoms + Q_TILE - 1) // Q_TILE) * Q_TILE
    nt = n_pad // Q_TILE

    ids32 = id_j.astype(jnp.int32)
    ids3 = ids32.reshape(nb, E_BLOCK, 1)

    tile_starts = jnp.searchsorted(
        ids32, jnp.arange(nt, dtype=jnp.int32) * Q_TILE).astype(jnp.int32)
    block_starts = jnp.arange(nb, dtype=jnp.int32) * E_BLOCK
    p = jnp.sort(jnp.concatenate([block_starts, tile_starts]))
    valid = (jnp.concatenate([jnp.ones((1,), jnp.bool_), p[1:] != p[:-1]])
             & (p < E)).astype(jnp.int32)
    pc = jnp.minimum(p, E - 1)
    eb = pc // E_BLOCK
    qt = ids32[pc] // Q_TILE

    n_chunks = nb + nt
    steps = (n_chunks + LANES - 1) // LANES
    pad = steps * LANES - n_chunks
    if pad:
        eb = jnp.concatenate([eb, jnp.broadcast_to(eb[-1], (pad,))])
        qt = jnp.concatenate([qt, jnp.broadcast_to(qt[-1], (pad,))])
        valid = jnp.concatenate([valid, jnp.zeros((pad,), jnp.int32)])
    ebL = eb.reshape(LANES, steps)
    qtL = qt.reshape(LANES, steps)
    vL = valid.reshape(LANES, steps)

    def _ids_map(s):
        return lambda c, e, q, v: (e[s, c], 0, 0)

    def _edge_map(s):
        return lambda c, e, q, v: (e[s, c], 0)

    grid_spec = pltpu.PrefetchScalarGridSpec(
        num_scalar_prefetch=3,
        grid=(steps,),
        in_specs=(
            [pl.BlockSpec((1, E_BLOCK, 1), _ids_map(s)) for s in range(LANES)]
            + [pl.BlockSpec((E_BLOCK, d_rbf), _edge_map(s))
               for s in range(LANES)]
            + [pl.BlockSpec((E_BLOCK, d_edge), _edge_map(s))
               for s in range(LANES)]
            + [pl.BlockSpec((d_rbf, d_edge), lambda c, e, q, v: (0, 0))]
        ),
        out_specs=pl.BlockSpec((n_pad, d_edge), lambda c, e, q, v: (0, 0)),
    )

    def _inner(qt_r, v_r, *refs):
        _scatter_kernel(qt_r, v_r, *refs)

    x2 = pl.pallas_call(
        lambda eb_r, qt_r, v_r, *refs: _scatter_kernel(qt_r, v_r, *refs),
        grid_spec=grid_spec,
        out_shape=jax.ShapeDtypeStruct((n_pad, d_edge), jnp.float32),
    )(ebL, qtL, vL, *([ids3] * LANES), *([rbf] * LANES), *([m] * LANES),
      W_rbf)

    w1s = W1 * scale_sum

    out = pl.pallas_call(
        _mlp_kernel,
        grid=(n_pad // R_BLOCK,),
        in_specs=[pl.BlockSpec((R_BLOCK, d_edge), lambda i: (i, 0))]
        + [pl.BlockSpec((D, D), lambda i: (0, 0))] * 5,
        out_specs=pl.BlockSpec((R_BLOCK, D), lambda i: (i, 0)),
        out_shape=jax.ShapeDtypeStruct((n_pad, D), jnp.float32),
    )(x2, w1s, W_res_0_0, W_res_0_1, W_res_1_0, W_res_1_1)

    return out[:nAtoms]
